# baseline (speedup 1.0000x reference)
import jax
import jax.numpy as jnp
from jax.experimental import pallas as pl
from jax.experimental.pallas import tpu as pltpu


def _copy_body(x_ref, o_ref, sem):
    copy = pltpu.make_async_copy(x_ref, o_ref, sem)
    copy.start()
    copy.wait()


def kernel(code_embeddings):
    return pl.pallas_call(
        _copy_body,
        out_shape=jax.ShapeDtypeStruct(code_embeddings.shape, code_embeddings.dtype),
        in_specs=[pl.BlockSpec(memory_space=pl.ANY)],
        out_specs=pl.BlockSpec(memory_space=pl.ANY),
        scratch_shapes=[pltpu.SemaphoreType.DMA],
    )(code_embeddings)


# speedup vs baseline: 9.0410x; 9.0410x over previous
import jax
import jax.numpy as jnp
from jax.experimental import pallas as pl
from jax.experimental.pallas import tpu as pltpu


def _copy_body(x_ref, o_ref):
    o_ref[...] = x_ref[...]


def kernel(code_embeddings):
    x = code_embeddings.reshape(50000, 128)
    n_blocks = 50
    out = pl.pallas_call(
        _copy_body,
        out_shape=jax.ShapeDtypeStruct((50000, 128), jnp.float32),
        grid=(n_blocks,),
        in_specs=[pl.BlockSpec((50000 // n_blocks, 128), lambda i: (i, 0))],
        out_specs=pl.BlockSpec((50000 // n_blocks, 128), lambda i: (i, 0)),
        compiler_params=pltpu.CompilerParams(
            dimension_semantics=("arbitrary",),
        ),
    )(x)
    return out.reshape(100000, 64)


# speedup vs baseline: 10.1267x; 1.1201x over previous
import jax
import jax.numpy as jnp
from jax.experimental import pallas as pl
from jax.experimental.pallas import tpu as pltpu

_ROWS = 50000
_NC = 50
_C = _ROWS // _NC


def _copy_body(x_hbm, o_hbm, vmem, in_sems, out_sems):
    def in_copy(i):
        return pltpu.make_async_copy(
            x_hbm.at[pl.ds(i * _C, _C)], vmem.at[i], in_sems.at[i])

    def out_copy(i):
        return pltpu.make_async_copy(
            vmem.at[i], o_hbm.at[pl.ds(i * _C, _C)], out_sems.at[i])

    for i in range(_NC):
        in_copy(i).start()
    for i in range(_NC):
        in_copy(i).wait()
        out_copy(i).start()
    for i in range(_NC):
        out_copy(i).wait()


def kernel(code_embeddings):
    x = code_embeddings.reshape(_ROWS, 128)
    out = pl.pallas_call(
        _copy_body,
        out_shape=jax.ShapeDtypeStruct((_ROWS, 128), jnp.float32),
        in_specs=[pl.BlockSpec(memory_space=pl.ANY)],
        out_specs=pl.BlockSpec(memory_space=pl.ANY),
        scratch_shapes=[
            pltpu.VMEM((_NC, _C, 128), jnp.float32),
            pltpu.SemaphoreType.DMA((_NC,)),
            pltpu.SemaphoreType.DMA((_NC,)),
        ],
    )(x)
    return out.reshape(100000, 64)


# speedup vs baseline: 15.4279x; 1.5235x over previous
import jax
import jax.numpy as jnp
from jax.experimental import pallas as pl
from jax.experimental.pallas import tpu as pltpu

_ROWS = 100000
_NC = 50
_C = _ROWS // _NC


def _copy_body(x_hbm, o_hbm, vmem, in_sems, out_sems):
    def in_copy(i):
        return pltpu.make_async_copy(
            x_hbm.at[pl.ds(i * _C, _C)], vmem.at[i], in_sems.at[i])

    def out_copy(i):
        return pltpu.make_async_copy(
            vmem.at[i], o_hbm.at[pl.ds(i * _C, _C)], out_sems.at[i])

    for i in range(_NC):
        in_copy(i).start()
    for i in range(_NC):
        in_copy(i).wait()
        out_copy(i).start()
    for i in range(_NC):
        out_copy(i).wait()


def kernel(code_embeddings):
    return pl.pallas_call(
        _copy_body,
        out_shape=jax.ShapeDtypeStruct((_ROWS, 64), jnp.float32),
        in_specs=[pl.BlockSpec(memory_space=pl.ANY)],
        out_specs=pl.BlockSpec(memory_space=pl.ANY),
        scratch_shapes=[
            pltpu.VMEM((_NC, _C, 64), jnp.float32),
            pltpu.SemaphoreType.DMA((_NC,)),
            pltpu.SemaphoreType.DMA((_NC,)),
        ],
    )(code_embeddings)
